# Initial kernel scaffold; baseline (speedup 1.0000x reference)
import jax
import jax.numpy as jnp
from jax.experimental import pallas as pl


def kernel(x, tables, num_weights, num_biases):
    raise NotImplementedError("write your pallas kernel here")


# speedup vs baseline: 1.1799x; 1.1799x over previous
import functools

import jax
import jax.numpy as jnp
from jax import lax
from jax.experimental import pallas as pl
from jax.experimental.pallas import tpu as pltpu
from jax.experimental.pallas import tpu_sc as plsc

_NUM = 13
_CAT = 26
_V = 100000
_D = 32

_NC = 2
_NS = 16
_NW = _NC * _NS
_NB = 128


def _sc_body(xcat_hbm, tab_hbm, out_hbm, xc_v, idx_v, gbuf, outb, sem):
  wid = lax.axis_index("s") * _NC + lax.axis_index("c")
  b = xcat_hbm.shape[1]
  b_per_w = b // _NW
  nblk = b_per_w // _NB

  def blk_body(blk, _):
    base = wid * b_per_w + blk * _NB
    pltpu.sync_copy(xcat_hbm.at[:, pl.ds(base, _NB)], xc_v)
    for c in range(_CAT):
      for j in range(_NB // 16):
        v = xc_v[c, pl.ds(j * 16, 16)]
        idx_v[c, pl.ds(j * 16, 16)] = v.astype(jnp.int32) + (c * _V)
    descs = []
    for c in range(_CAT):
      descs.append(
          pltpu.async_copy(tab_hbm.at[idx_v.at[c]], gbuf.at[c], sem))
    for d in descs:
      d.wait()
    def row_body(r, _):
      a0 = gbuf[0, r, pl.ds(0, 16)]
      a1 = gbuf[0, r, pl.ds(16, 16)]
      for c in range(1, _CAT):
        a0 = a0 + gbuf[c, r, pl.ds(0, 16)]
        a1 = a1 + gbuf[c, r, pl.ds(16, 16)]
      outb[r, pl.ds(0, 16)] = a0
      outb[r, pl.ds(16, 16)] = a1
      return 0
    lax.fori_loop(0, _NB, row_body, 0)
    pltpu.sync_copy(outb, out_hbm.at[pl.ds(base, _NB)])
    return 0

  lax.fori_loop(0, nblk, blk_body, 0)


def _sc_gather_sum(x_cat_t, tab_flat):
  b = x_cat_t.shape[1]
  mesh = plsc.VectorSubcoreMesh(core_axis_name="c", subcore_axis_name="s")
  return pl.kernel(
      _sc_body,
      out_type=jax.ShapeDtypeStruct((b, _D), jnp.float32),
      mesh=mesh,
      scratch_types=[
          pltpu.VMEM((_CAT, _NB), jnp.float32),
          pltpu.VMEM((_CAT, _NB), jnp.int32),
          pltpu.VMEM((_CAT, _NB, _D), jnp.float32),
          pltpu.VMEM((_NB, _D), jnp.float32),
          pltpu.SemaphoreType.DMA,
      ],
      compiler_params=pltpu.CompilerParams(use_tc_tiling_on_sc=False),
  )(x_cat_t, tab_flat)


def _tc_body(xn_ref, w_ref, b_ref, cs_ref, o_ref):
  xn = xn_ref[...]
  w = w_ref[...]
  bias_sum = jnp.sum(b_ref[...], axis=0, keepdims=True)
  num = jnp.dot(xn, w, preferred_element_type=jnp.float32)
  o_ref[...] = (num + bias_sum + cs_ref[...]) * (1.0 / (_NUM + _CAT))


def _tc_finalize(x_num, num_weights, num_biases, cat_sum):
  b = x_num.shape[0]
  bt = 4096
  grid = b // bt
  return pl.pallas_call(
      _tc_body,
      grid=(grid,),
      in_specs=[
          pl.BlockSpec((bt, _NUM), lambda i: (i, 0)),
          pl.BlockSpec((_NUM, _D), lambda i: (0, 0)),
          pl.BlockSpec((_NUM, _D), lambda i: (0, 0)),
          pl.BlockSpec((bt, _D), lambda i: (i, 0)),
      ],
      out_specs=pl.BlockSpec((bt, _D), lambda i: (i, 0)),
      out_shape=jax.ShapeDtypeStruct((b, _D), jnp.float32),
  )(x_num, num_weights, num_biases, cat_sum)


@jax.jit
def kernel(x, tables, num_weights, num_biases):
  x_num = x[:, :_NUM]
  x_cat_t = x[:, _NUM:].T
  tab_flat = tables.reshape(_CAT * _V, _D)
  cat_sum = _sc_gather_sum(x_cat_t, tab_flat)
  return _tc_finalize(x_num, num_weights, num_biases, cat_sum)
